# baseline (speedup 1.0000x reference)
import functools

import jax
import jax.numpy as jnp
from jax.experimental import pallas as pl
from jax.experimental.pallas import tpu as pltpu

_T = 512
_TSHIFT = 9
_C = 256


def _round_up(x, m):
    return ((x + m - 1) // m) * m


def _pad2d(a, rows, cols):
    if a.shape == (rows, cols):
        return a
    return jnp.pad(a, ((0, rows - a.shape[0]), (0, cols - a.shape[1])))


def _proj_kernel(x_ref, w_ref, h_ref):
    h_ref[...] = jnp.dot(x_ref[...], w_ref[...],
                         preferred_element_type=jnp.float32).astype(h_ref.dtype)


def _chunk_onehots(src_ref, dst_ref, tk):
    sv = src_ref[0]
    dv = dst_ref[0, 0]
    lane = jax.lax.broadcasted_iota(jnp.int32, (_C, tk), 1)
    oh_s = (sv == lane).astype(jnp.bfloat16)
    row = jax.lax.broadcasted_iota(jnp.int32, (_T, _C), 0)
    oh_d = (row == dv).astype(jnp.bfloat16)
    return oh_s, oh_d


def _agg1_kernel(tile_r, kblk_r, len_r, first_r, last_r,
                 src_ref, dst_ref, hp_ref, x_ref, wr_ref, b_ref, w2_ref,
                 h1_ref, h2p_ref, invd_ref, acc_ref, dacc_ref):
    c = pl.program_id(0)
    g = pl.program_id(1)

    @pl.when(first_r[c, g] == 1)
    def _():
        acc_ref[...] = jnp.zeros_like(acc_ref)
        dacc_ref[...] = jnp.zeros_like(dacc_ref)

    @pl.when(len_r[c, g] > 0)
    def _():
        oh_s, oh_d = _chunk_onehots(src_ref, dst_ref, _T)
        koff = pl.multiple_of(kblk_r[c, g] * _T, _T)
        grows = jnp.dot(oh_s, hp_ref[pl.ds(koff, _T), :],
                        preferred_element_type=jnp.float32)
        acc_ref[...] += jnp.dot(oh_d, grows.astype(jnp.bfloat16),
                                preferred_element_type=jnp.float32)
        dacc_ref[...] += jnp.sum(oh_d, axis=1, keepdims=True
                                 ).astype(jnp.float32)

    @pl.when(last_r[c, g] == 1)
    def _():
        deg = dacc_ref[...]
        inv = jnp.where(deg > 0, 1.0 / deg, 0.0)
        invd_ref[...] = inv
        self_term = jnp.dot(x_ref[...], wr_ref[...],
                            preferred_element_type=jnp.float32) + b_ref[...]
        h1 = jnp.maximum(acc_ref[...] * inv + self_term, 0.0)
        h1_bf = h1.astype(jnp.bfloat16)
        h1_ref[...] = h1_bf
        h2p_ref[...] = jnp.dot(h1_bf, w2_ref[...],
                               preferred_element_type=jnp.float32
                               ).astype(h2p_ref.dtype)


def _agg2_kernel(tile_r, kblk_r, len_r, first_r, last_r,
                 src_ref, dst_ref, hp_ref, h1_ref, wr_ref, b_ref, inv_ref,
                 o_ref, acc_ref, *, n_classes):
    c = pl.program_id(0)
    g = pl.program_id(1)

    @pl.when(first_r[c, g] == 1)
    def _():
        acc_ref[...] = jnp.zeros_like(acc_ref)

    @pl.when(len_r[c, g] > 0)
    def _():
        oh_s, oh_d = _chunk_onehots(src_ref, dst_ref, _T)
        koff = pl.multiple_of(kblk_r[c, g] * _T, _T)
        grows = jnp.dot(oh_s, hp_ref[pl.ds(koff, _T), :],
                        preferred_element_type=jnp.float32)
        acc_ref[...] += jnp.dot(oh_d, grows.astype(jnp.bfloat16),
                                preferred_element_type=jnp.float32)

    @pl.when(last_r[c, g] == 1)
    def _():
        self_term = jnp.dot(h1_ref[...], wr_ref[...],
                            preferred_element_type=jnp.float32) + b_ref[...]
        out = acc_ref[...] * inv_ref[...] + self_term
        col = jax.lax.broadcasted_iota(jnp.int32, out.shape, 1)
        out = jnp.where(col < n_classes, out, -jnp.inf)
        m = jnp.max(out, axis=1, keepdims=True)
        shifted = out - m
        lse = jnp.log(jnp.sum(jnp.exp(shifted), axis=1, keepdims=True))
        o_ref[...] = (shifted - lse).astype(o_ref.dtype)


def _chunk_schedule(edge_index, n_pad):
    e = edge_index.shape[1]
    n_t = n_pad // _T
    n_b = n_t * n_t
    g_half = e // _C + n_b + n_t + 1

    src, dst = edge_index[0], edge_index[1]
    mask = jnp.int32(_T - 1)
    blk = (dst >> _TSHIFT) * n_t + (src >> _TSHIFT)
    key = (blk << (2 * _TSHIFT)) | ((dst & mask) << _TSHIFT) | (src & mask)
    ks = jnp.sort(key)

    bounds = (jnp.arange(n_b + 1, dtype=jnp.int32) << (2 * _TSHIFT))
    bnd = jnp.searchsorted(ks, bounds, side="left").astype(jnp.int32)
    blk_start = bnd[:-1]
    cnt = bnd[1:] - bnd[:-1]

    c_b = (cnt + _C - 1) // _C
    per_tile = c_b.reshape(n_t, n_t)
    fix = (per_tile.sum(axis=1) == 0).astype(jnp.int32)
    col0 = (jnp.arange(n_t, dtype=jnp.int32)[None, :] == 0).astype(jnp.int32)
    c_b = (per_tile + fix[:, None] * col0).reshape(-1)

    chunk_excl = jnp.concatenate(
        [jnp.zeros((1,), jnp.int32), jnp.cumsum(c_b).astype(jnp.int32)])
    total = chunk_excl[-1]

    g_glob = g_half
    gidx = jnp.arange(g_glob, dtype=jnp.int32)
    blk_of = jnp.minimum(jnp.searchsorted(chunk_excl[1:], gidx, side="right"
                                          ).astype(jnp.int32), n_b - 1)
    rank = gidx - chunk_excl[blk_of]
    start_g = blk_start[blk_of] + rank * _C
    len_g = jnp.clip(cnt[blk_of] - rank * _C, 0, _C)
    tile_g = blk_of // n_t
    kblk_g = blk_of % n_t

    tile_chunks = c_b.reshape(n_t, n_t).sum(axis=1)
    cum = jnp.cumsum(tile_chunks).astype(jnp.int32)
    s = jnp.clip(jnp.searchsorted(cum, total // 2, side="left"),
                 0, n_t - 2).astype(jnp.int32)
    cs = cum[s]

    g = jnp.arange(g_half, dtype=jnp.int32)
    idx0 = jnp.clip(g, 0, cs - 1)
    idx1 = jnp.clip(cs + g, 0, total - 1)
    real = jnp.stack([g < cs, (cs + g) < total])
    idx = jnp.stack([idx0, idx1])

    tile_h = tile_g[idx]
    kblk_h = kblk_g[idx]
    len_h = jnp.where(real, len_g[idx], 0).astype(jnp.int32)
    start_h = start_g[idx]

    first_h = jnp.concatenate(
        [jnp.ones((2, 1), jnp.int32),
         (tile_h[:, 1:] != tile_h[:, :-1]).astype(jnp.int32)], axis=1)
    last_h = jnp.concatenate(
        [(tile_h[:, 1:] != tile_h[:, :-1]).astype(jnp.int32),
         jnp.ones((2, 1), jnp.int32)], axis=1)

    valid = jnp.arange(_C, dtype=jnp.int32)[None, None, :] < len_h[:, :, None]
    ks_pad = jnp.concatenate([ks, jnp.zeros((_C,), jnp.int32)])
    starts = jnp.clip(start_h.reshape(-1), 0, e - 1)
    keys_c = jax.vmap(
        lambda s: jax.lax.dynamic_slice(ks_pad, (s,), (_C,)))(starts)
    keys_c = keys_c.reshape(2, g_half, _C)
    src_l = jnp.where(valid, keys_c & mask, -1).astype(jnp.int32)
    dst_l = jnp.where(valid, (keys_c >> _TSHIFT) & mask, -1).astype(jnp.int32)

    return (tile_h.astype(jnp.int32), kblk_h.astype(jnp.int32), len_h,
            first_h, last_h,
            src_l.reshape(2, g_half * _C, 1),
            dst_l.reshape(2, g_half, 1, _C),
            g_half)


def kernel(x, edge_index, conv0_w_l, conv0_w_r, conv0_b_l,
           out_w_l, out_w_r, out_b_l):
    n, f_in = x.shape
    f_hid = conv0_w_l.shape[1]
    n_classes = out_w_l.shape[1]

    n_pad = _round_up(n, _T)
    f_in_p = _round_up(f_in, 128)
    f_hid_p = _round_up(f_hid, 128)
    f_out_p = _round_up(n_classes, 128)
    n_rows = n_pad // _T

    (tile_h, kblk_h, len_h, first_h, last_h, src_l, dst_l,
     g_half) = _chunk_schedule(edge_index, n_pad)

    xb = _pad2d(x, n_pad, f_in_p).astype(jnp.bfloat16)
    w1l = _pad2d(conv0_w_l, f_in_p, f_hid_p).astype(jnp.bfloat16)
    w1r = _pad2d(conv0_w_r, f_in_p, f_hid_p).astype(jnp.bfloat16)
    b1 = _pad2d(conv0_b_l, 1, f_hid_p)
    w2l = _pad2d(out_w_l, f_hid_p, f_out_p).astype(jnp.bfloat16)
    w2r = _pad2d(out_w_r, f_hid_p, f_out_p).astype(jnp.bfloat16)
    b2 = _pad2d(out_b_l, 1, f_out_p)

    h1p = pl.pallas_call(
        _proj_kernel,
        out_shape=jax.ShapeDtypeStruct((n_pad, f_hid_p), jnp.bfloat16),
        grid=(n_rows,),
        in_specs=[
            pl.BlockSpec((_T, f_in_p), lambda i: (i, 0)),
            pl.BlockSpec((f_in_p, f_hid_p), lambda i: (0, 0)),
        ],
        out_specs=pl.BlockSpec((_T, f_hid_p), lambda i: (i, 0)),
        compiler_params=pltpu.CompilerParams(
            dimension_semantics=("parallel",)),
    )(xb, w1l)

    cparams = pltpu.CompilerParams(
        dimension_semantics=("parallel", "arbitrary"),
        vmem_limit_bytes=48 * 1024 * 1024,
    )

    h1, h2p, inv_deg = pl.pallas_call(
        _agg1_kernel,
        out_shape=(
            jax.ShapeDtypeStruct((n_pad, f_hid_p), jnp.bfloat16),
            jax.ShapeDtypeStruct((n_pad, f_out_p), jnp.bfloat16),
            jax.ShapeDtypeStruct((n_pad, 1), jnp.float32),
        ),
        grid_spec=pltpu.PrefetchScalarGridSpec(
            num_scalar_prefetch=5,
            grid=(2, g_half),
            in_specs=[
                pl.BlockSpec((1, _C, 1),
                             lambda c, g, t, k, l, f, la: (c, g, 0)),
                pl.BlockSpec((1, 1, 1, _C),
                             lambda c, g, t, k, l, f, la: (c, g, 0, 0)),
                pl.BlockSpec((n_pad, f_hid_p),
                             lambda c, g, t, k, l, f, la: (0, 0)),
                pl.BlockSpec((_T, f_in_p),
                             lambda c, g, t, k, l, f, la: (t[c, g], 0)),
                pl.BlockSpec((f_in_p, f_hid_p),
                             lambda c, g, t, k, l, f, la: (0, 0)),
                pl.BlockSpec((1, f_hid_p),
                             lambda c, g, t, k, l, f, la: (0, 0)),
                pl.BlockSpec((f_hid_p, f_out_p),
                             lambda c, g, t, k, l, f, la: (0, 0)),
            ],
            out_specs=(
                pl.BlockSpec((_T, f_hid_p),
                             lambda c, g, t, k, l, f, la: (t[c, g], 0)),
                pl.BlockSpec((_T, f_out_p),
                             lambda c, g, t, k, l, f, la: (t[c, g], 0)),
                pl.BlockSpec((_T, 1),
                             lambda c, g, t, k, l, f, la: (t[c, g], 0)),
            ),
            scratch_shapes=[pltpu.VMEM((_T, f_hid_p), jnp.float32),
                            pltpu.VMEM((_T, 1), jnp.float32)],
        ),
        compiler_params=cparams,
    )(tile_h, kblk_h, len_h, first_h, last_h, src_l, dst_l,
      h1p, xb, w1r, b1, w2l)

    out = pl.pallas_call(
        functools.partial(_agg2_kernel, n_classes=n_classes),
        out_shape=jax.ShapeDtypeStruct((n_pad, f_out_p), jnp.float32),
        grid_spec=pltpu.PrefetchScalarGridSpec(
            num_scalar_prefetch=5,
            grid=(2, g_half),
            in_specs=[
                pl.BlockSpec((1, _C, 1),
                             lambda c, g, t, k, l, f, la: (c, g, 0)),
                pl.BlockSpec((1, 1, 1, _C),
                             lambda c, g, t, k, l, f, la: (c, g, 0, 0)),
                pl.BlockSpec((n_pad, f_out_p),
                             lambda c, g, t, k, l, f, la: (0, 0)),
                pl.BlockSpec((_T, f_hid_p),
                             lambda c, g, t, k, l, f, la: (t[c, g], 0)),
                pl.BlockSpec((f_hid_p, f_out_p),
                             lambda c, g, t, k, l, f, la: (0, 0)),
                pl.BlockSpec((1, f_out_p),
                             lambda c, g, t, k, l, f, la: (0, 0)),
                pl.BlockSpec((_T, 1),
                             lambda c, g, t, k, l, f, la: (t[c, g], 0)),
            ],
            out_specs=pl.BlockSpec((_T, f_out_p),
                                   lambda c, g, t, k, l, f, la: (t[c, g], 0)),
            scratch_shapes=[pltpu.VMEM((_T, f_out_p), jnp.float32)],
        ),
        compiler_params=cparams,
    )(tile_h, kblk_h, len_h, first_h, last_h, src_l, dst_l,
      h2p, h1, w2r, b2, inv_deg)

    return out[:n, :n_classes]


# speedup vs baseline: 4.0359x; 4.0359x over previous
import functools

import jax
import jax.numpy as jnp
from jax.experimental import pallas as pl
from jax.experimental.pallas import tpu as pltpu

_T = 512
_TSHIFT = 9
_L = 128


def _round_up(x, m):
    return ((x + m - 1) // m) * m


def _pad2d(a, rows, cols):
    if a.shape == (rows, cols):
        return a
    return jnp.pad(a, ((0, rows - a.shape[0]), (0, cols - a.shape[1])))


def _proj_kernel(x_ref, w_ref, h_ref):
    h_ref[...] = jnp.dot(x_ref[...], w_ref[...],
                         preferred_element_type=jnp.float32).astype(h_ref.dtype)


def _unit_onehots(ks_ref, m1, m2):
    row = m1 >> 10
    lo = (m2 >> 2) & 0x7F
    hi = (m2 >> 9) & 0xFF
    base = pl.multiple_of((row >> 3) << 3, 8)
    slab = ks_ref[pl.ds(base, 8), :]
    kv = pltpu.roll(slab, -(row & 7), axis=0)[0:1, :]
    lane = jax.lax.broadcasted_iota(jnp.int32, (1, _L), 1)
    msk = (lane >= lo) & (lane < hi)
    mask_i = jnp.int32(_T - 1)
    srcv = jnp.where(msk, kv & mask_i, -1)
    dstv = jnp.where(msk, (kv >> _TSHIFT) & mask_i, -1)
    rowio = jax.lax.broadcasted_iota(jnp.int32, (_T, _L), 0)
    oh_s = (rowio == srcv).astype(jnp.bfloat16)
    oh_d = (rowio == dstv).astype(jnp.bfloat16)
    return oh_s, oh_d


def _gather_scatter(ks_ref, hp_ref, m1, m2):
    oh_s, oh_d = _unit_onehots(ks_ref, m1, m2)
    koff = pl.multiple_of(((m1 >> 5) & 31) * _T, _T)
    grows = jax.lax.dot_general(
        oh_s, hp_ref[pl.ds(koff, _T), :],
        dimension_numbers=(((0,), (0,)), ((), ())),
        preferred_element_type=jnp.float32)
    contrib = jnp.dot(oh_d, grows.astype(jnp.bfloat16),
                      preferred_element_type=jnp.float32)
    return contrib, oh_d


def _agg1_kernel(m1_r, m2_r, ks_ref, hp_ref, x_ref, wr_ref, b_ref, w2_ref,
                 h1_ref, h2p_ref, invd_ref, acc_ref, dacc_ref):
    c = pl.program_id(0)
    g = pl.program_id(1)
    m1 = m1_r[c, g]
    m2 = m2_r[c, g]

    @pl.when(m2 & 1 == 1)
    def _():
        acc_ref[...] = jnp.zeros_like(acc_ref)
        dacc_ref[...] = jnp.zeros_like(dacc_ref)

    @pl.when(((m2 >> 2) & 0x7F) < ((m2 >> 9) & 0xFF))
    def _():
        contrib, oh_d = _gather_scatter(ks_ref, hp_ref, m1, m2)
        acc_ref[...] += contrib
        dacc_ref[...] += jnp.sum(oh_d, axis=1, keepdims=True
                                 ).astype(jnp.float32)

    @pl.when((m2 >> 1) & 1 == 1)
    def _():
        deg = dacc_ref[...]
        inv = jnp.where(deg > 0, 1.0 / deg, 0.0)
        invd_ref[...] = inv
        self_term = jnp.dot(x_ref[...], wr_ref[...],
                            preferred_element_type=jnp.float32) + b_ref[...]
        h1 = jnp.maximum(acc_ref[...] * inv + self_term, 0.0)
        h1_bf = h1.astype(jnp.bfloat16)
        h1_ref[...] = h1_bf
        h2p_ref[...] = jnp.dot(h1_bf, w2_ref[...],
                               preferred_element_type=jnp.float32
                               ).astype(h2p_ref.dtype)


def _agg2_kernel(m1_r, m2_r, ks_ref, hp_ref, h1_ref, wr_ref, b_ref, inv_ref,
                 o_ref, acc_ref, *, n_classes):
    c = pl.program_id(0)
    g = pl.program_id(1)
    m1 = m1_r[c, g]
    m2 = m2_r[c, g]

    @pl.when(m2 & 1 == 1)
    def _():
        acc_ref[...] = jnp.zeros_like(acc_ref)

    @pl.when(((m2 >> 2) & 0x7F) < ((m2 >> 9) & 0xFF))
    def _():
        contrib, _ = _gather_scatter(ks_ref, hp_ref, m1, m2)
        acc_ref[...] += contrib

    @pl.when((m2 >> 1) & 1 == 1)
    def _():
        self_term = jnp.dot(h1_ref[...], wr_ref[...],
                            preferred_element_type=jnp.float32) + b_ref[...]
        out = acc_ref[...] * inv_ref[...] + self_term
        col = jax.lax.broadcasted_iota(jnp.int32, out.shape, 1)
        out = jnp.where(col < n_classes, out, -jnp.inf)
        m = jnp.max(out, axis=1, keepdims=True)
        shifted = out - m
        lse = jnp.log(jnp.sum(jnp.exp(shifted), axis=1, keepdims=True))
        o_ref[...] = (shifted - lse).astype(o_ref.dtype)


def _unit_schedule(edge_index, n_pad):
    e = edge_index.shape[1]
    n_t = n_pad // _T
    n_b = n_t * n_t
    g_half = e // _L + n_b + n_t + 1

    src, dst = edge_index[0], edge_index[1]
    mask = jnp.int32(_T - 1)
    blk = (dst >> _TSHIFT) * n_t + (src >> _TSHIFT)
    key = (blk << (2 * _TSHIFT)) | ((dst & mask) << _TSHIFT) | (src & mask)
    ks = jnp.sort(key)

    bounds = (jnp.arange(n_b + 1, dtype=jnp.int32) << (2 * _TSHIFT))
    bnd = jnp.searchsorted(ks, bounds, side="left").astype(jnp.int32)
    blk_start = bnd[:-1]
    cnt = bnd[1:] - bnd[:-1]

    u_b = jnp.where(
        cnt > 0,
        ((blk_start + cnt - 1) >> 7) - (blk_start >> 7) + 1,
        0).astype(jnp.int32)
    per_tile = u_b.reshape(n_t, n_t)
    fix = (per_tile.sum(axis=1) == 0).astype(jnp.int32)
    col0 = (jnp.arange(n_t, dtype=jnp.int32)[None, :] == 0).astype(jnp.int32)
    u_b = (per_tile + fix[:, None] * col0).reshape(-1)

    u_excl = jnp.concatenate(
        [jnp.zeros((1,), jnp.int32), jnp.cumsum(u_b).astype(jnp.int32)])
    total = u_excl[-1]
    uidx = jnp.arange(g_half, dtype=jnp.int32)
    blk_of = jnp.minimum(jnp.searchsorted(u_excl[1:], uidx, side="right"
                                          ).astype(jnp.int32), n_b - 1)
    rank = uidx - u_excl[blk_of]
    row_u = (blk_start[blk_of] >> 7) + rank
    lo_u = jnp.clip(blk_start[blk_of] - (row_u << 7), 0, _L)
    hi_u = jnp.clip(blk_start[blk_of] + cnt[blk_of] - (row_u << 7), 0, _L)
    tile_u = blk_of // n_t
    kblk_u = blk_of % n_t

    tile_units = u_b.reshape(n_t, n_t).sum(axis=1)
    cum = jnp.cumsum(tile_units).astype(jnp.int32)
    s = jnp.clip(jnp.searchsorted(cum, total // 2, side="left"),
                 0, n_t - 2).astype(jnp.int32)
    cs = cum[s]

    g = jnp.arange(g_half, dtype=jnp.int32)
    idx0 = jnp.clip(g, 0, cs - 1)
    idx1 = jnp.clip(cs + g, 0, total - 1)
    real = jnp.stack([g < cs, (cs + g) < total])
    idx = jnp.stack([idx0, idx1])

    tile_h = tile_u[idx]
    kblk_h = kblk_u[idx]
    row_h = row_u[idx]
    lo_h = jnp.where(real, lo_u[idx], 0)
    hi_h = jnp.where(real, hi_u[idx], 0)

    first_h = jnp.concatenate(
        [jnp.ones((2, 1), jnp.int32),
         (tile_h[:, 1:] != tile_h[:, :-1]).astype(jnp.int32)], axis=1)
    last_h = jnp.concatenate(
        [(tile_h[:, 1:] != tile_h[:, :-1]).astype(jnp.int32),
         jnp.ones((2, 1), jnp.int32)], axis=1)

    m1 = (row_h << 10) | (kblk_h << 5) | tile_h
    m2 = first_h | (last_h << 1) | (lo_h << 2) | (hi_h << 9)

    r_real = max((e + _L - 1) // _L, 1)
    r_pad = _round_up(r_real, 8) + 8
    ks2 = jnp.pad(ks, (0, r_pad * _L - e)).reshape(r_pad, _L)

    return m1.astype(jnp.int32), m2.astype(jnp.int32), ks2, g_half


def kernel(x, edge_index, conv0_w_l, conv0_w_r, conv0_b_l,
           out_w_l, out_w_r, out_b_l):
    n, f_in = x.shape
    f_hid = conv0_w_l.shape[1]
    n_classes = out_w_l.shape[1]

    n_pad = _round_up(n, _T)
    f_in_p = _round_up(f_in, 128)
    f_hid_p = _round_up(f_hid, 128)
    f_out_p = _round_up(n_classes, 128)
    n_rows = n_pad // _T

    m1, m2, ks2, g_half = _unit_schedule(edge_index, n_pad)
    r_pad = ks2.shape[0]

    xb = _pad2d(x, n_pad, f_in_p).astype(jnp.bfloat16)
    w1l = _pad2d(conv0_w_l, f_in_p, f_hid_p).astype(jnp.bfloat16)
    w1r = _pad2d(conv0_w_r, f_in_p, f_hid_p).astype(jnp.bfloat16)
    b1 = _pad2d(conv0_b_l, 1, f_hid_p)
    w2l = _pad2d(out_w_l, f_hid_p, f_out_p).astype(jnp.bfloat16)
    w2r = _pad2d(out_w_r, f_hid_p, f_out_p).astype(jnp.bfloat16)
    b2 = _pad2d(out_b_l, 1, f_out_p)

    h1p = pl.pallas_call(
        _proj_kernel,
        out_shape=jax.ShapeDtypeStruct((n_pad, f_hid_p), jnp.bfloat16),
        grid=(n_rows,),
        in_specs=[
            pl.BlockSpec((_T, f_in_p), lambda i: (i, 0)),
            pl.BlockSpec((f_in_p, f_hid_p), lambda i: (0, 0)),
        ],
        out_specs=pl.BlockSpec((_T, f_hid_p), lambda i: (i, 0)),
        compiler_params=pltpu.CompilerParams(
            dimension_semantics=("parallel",)),
    )(xb, w1l)

    cparams = pltpu.CompilerParams(
        dimension_semantics=("parallel", "arbitrary"),
        vmem_limit_bytes=48 * 1024 * 1024,
    )

    h1, h2p, inv_deg = pl.pallas_call(
        _agg1_kernel,
        out_shape=(
            jax.ShapeDtypeStruct((n_pad, f_hid_p), jnp.bfloat16),
            jax.ShapeDtypeStruct((n_pad, f_out_p), jnp.bfloat16),
            jax.ShapeDtypeStruct((n_pad, 1), jnp.float32),
        ),
        grid_spec=pltpu.PrefetchScalarGridSpec(
            num_scalar_prefetch=2,
            grid=(2, g_half),
            in_specs=[
                pl.BlockSpec((r_pad, _L), lambda c, g, m1r, m2r: (0, 0)),
                pl.BlockSpec((n_pad, f_hid_p), lambda c, g, m1r, m2r: (0, 0)),
                pl.BlockSpec((_T, f_in_p),
                             lambda c, g, m1r, m2r: (m1r[c, g] & 31, 0)),
                pl.BlockSpec((f_in_p, f_hid_p), lambda c, g, m1r, m2r: (0, 0)),
                pl.BlockSpec((1, f_hid_p), lambda c, g, m1r, m2r: (0, 0)),
                pl.BlockSpec((f_hid_p, f_out_p), lambda c, g, m1r, m2r: (0, 0)),
            ],
            out_specs=(
                pl.BlockSpec((_T, f_hid_p),
                             lambda c, g, m1r, m2r: (m1r[c, g] & 31, 0)),
                pl.BlockSpec((_T, f_out_p),
                             lambda c, g, m1r, m2r: (m1r[c, g] & 31, 0)),
                pl.BlockSpec((_T, 1),
                             lambda c, g, m1r, m2r: (m1r[c, g] & 31, 0)),
            ),
            scratch_shapes=[pltpu.VMEM((_T, f_hid_p), jnp.float32),
                            pltpu.VMEM((_T, 1), jnp.float32)],
        ),
        compiler_params=cparams,
    )(m1, m2, ks2, h1p, xb, w1r, b1, w2l)

    out = pl.pallas_call(
        functools.partial(_agg2_kernel, n_classes=n_classes),
        out_shape=jax.ShapeDtypeStruct((n_pad, f_out_p), jnp.float32),
        grid_spec=pltpu.PrefetchScalarGridSpec(
            num_scalar_prefetch=2,
            grid=(2, g_half),
            in_specs=[
                pl.BlockSpec((r_pad, _L), lambda c, g, m1r, m2r: (0, 0)),
                pl.BlockSpec((n_pad, f_out_p), lambda c, g, m1r, m2r: (0, 0)),
                pl.BlockSpec((_T, f_hid_p),
                             lambda c, g, m1r, m2r: (m1r[c, g] & 31, 0)),
                pl.BlockSpec((f_hid_p, f_out_p), lambda c, g, m1r, m2r: (0, 0)),
                pl.BlockSpec((1, f_out_p), lambda c, g, m1r, m2r: (0, 0)),
                pl.BlockSpec((_T, 1),
                             lambda c, g, m1r, m2r: (m1r[c, g] & 31, 0)),
            ],
            out_specs=pl.BlockSpec((_T, f_out_p),
                                   lambda c, g, m1r, m2r: (m1r[c, g] & 31, 0)),
            scratch_shapes=[pltpu.VMEM((_T, f_out_p), jnp.float32)],
        ),
        compiler_params=cparams,
    )(m1, m2, ks2, h2p, h1, w2r, b2, inv_deg)

    return out[:n, :n_classes]


# speedup vs baseline: 7.5432x; 1.8690x over previous
import functools

import jax
import jax.numpy as jnp
from jax.experimental import pallas as pl
from jax.experimental.pallas import tpu as pltpu

_T = 512
_TSHIFT = 9
_L = 128
_U = 8


def _round_up(x, m):
    return ((x + m - 1) // m) * m


def _pad2d(a, rows, cols):
    if a.shape == (rows, cols):
        return a
    return jnp.pad(a, ((0, rows - a.shape[0]), (0, cols - a.shape[1])))


def _proj_kernel(x_ref, w_ref, h_ref):
    h_ref[...] = jnp.dot(x_ref[...], w_ref[...],
                         preferred_element_type=jnp.float32).astype(h_ref.dtype)


def _unit_onehots(ks_ref, m1, m2):
    row = m1 >> 10
    lo = (m2 >> 2) & 0x7F
    hi = (m2 >> 9) & 0xFF
    base = pl.multiple_of((row >> 3) << 3, 8)
    slab = ks_ref[pl.ds(base, 8), :]
    kv = pltpu.roll(slab, -(row & 7), axis=0)[0:1, :]
    lane = jax.lax.broadcasted_iota(jnp.int32, (1, _L), 1)
    msk = (lane >= lo) & (lane < hi)
    mask_i = jnp.int32(_T - 1)
    srcv = jnp.where(msk, kv & mask_i, -1)
    dstv = jnp.where(msk, (kv >> _TSHIFT) & mask_i, -1)
    rowio = jax.lax.broadcasted_iota(jnp.int32, (_T, _L), 0)
    oh_s = (rowio == srcv).astype(jnp.bfloat16)
    oh_d = (rowio == dstv).astype(jnp.bfloat16)
    return oh_s, oh_d


def _gather_scatter(ks_ref, hp_ref, m1, m2):
    oh_s, oh_d = _unit_onehots(ks_ref, m1, m2)
    koff = pl.multiple_of(((m1 >> 5) & 31) * _T, _T)
    grows = jax.lax.dot_general(
        oh_s, hp_ref[pl.ds(koff, _T), :],
        dimension_numbers=(((0,), (0,)), ((), ())),
        preferred_element_type=jnp.float32)
    contrib = jnp.dot(oh_d, grows.astype(jnp.bfloat16),
                      preferred_element_type=jnp.float32)
    return contrib, oh_d


def _agg1_kernel(m1_r, m2_r, nv_r, ks_ref, hp_ref, x_ref, wr_ref, b_ref,
                 w2_ref, h1_ref, h2p_ref, invd_ref, acc_ref, dacc_ref):
    c = pl.program_id(0)
    g = pl.program_id(1)
    nv = nv_r[c, g]

    @pl.when(nv & 16 == 16)
    def _():
        acc_ref[...] = jnp.zeros_like(acc_ref)
        dacc_ref[...] = jnp.zeros_like(dacc_ref)

    @pl.when(nv & 15 > 0)
    def _():
        total = None
        drow = None
        for u in range(_U):
            m1 = m1_r[c, g * _U + u]
            m2 = m2_r[c, g * _U + u]
            contrib, oh_d = _gather_scatter(ks_ref, hp_ref, m1, m2)
            d = jnp.sum(oh_d, axis=1, keepdims=True).astype(jnp.float32)
            total = contrib if total is None else total + contrib
            drow = d if drow is None else drow + d
        acc_ref[...] += total
        dacc_ref[...] += drow

    @pl.when(nv & 32 == 32)
    def _():
        deg = dacc_ref[...]
        inv = jnp.where(deg > 0, 1.0 / deg, 0.0)
        invd_ref[...] = inv
        self_term = jnp.dot(x_ref[...], wr_ref[...],
                            preferred_element_type=jnp.float32) + b_ref[...]
        h1 = jnp.maximum(acc_ref[...] * inv + self_term, 0.0)
        h1_bf = h1.astype(jnp.bfloat16)
        h1_ref[...] = h1_bf
        h2p_ref[...] = jnp.dot(h1_bf, w2_ref[...],
                               preferred_element_type=jnp.float32
                               ).astype(h2p_ref.dtype)


def _agg2_kernel(m1_r, m2_r, nv_r, ks_ref, hp_ref, h1_ref, wr_ref, b_ref,
                 inv_ref, o_ref, acc_ref, *, n_classes):
    c = pl.program_id(0)
    g = pl.program_id(1)
    nv = nv_r[c, g]

    @pl.when(nv & 16 == 16)
    def _():
        acc_ref[...] = jnp.zeros_like(acc_ref)

    @pl.when(nv & 15 > 0)
    def _():
        total = None
        for u in range(_U):
            m1 = m1_r[c, g * _U + u]
            m2 = m2_r[c, g * _U + u]
            contrib, _ = _gather_scatter(ks_ref, hp_ref, m1, m2)
            total = contrib if total is None else total + contrib
        acc_ref[...] += total

    @pl.when(nv & 32 == 32)
    def _():
        self_term = jnp.dot(h1_ref[...], wr_ref[...],
                            preferred_element_type=jnp.float32) + b_ref[...]
        out = acc_ref[...] * inv_ref[...] + self_term
        col = jax.lax.broadcasted_iota(jnp.int32, out.shape, 1)
        out = jnp.where(col < n_classes, out, -jnp.inf)
        m = jnp.max(out, axis=1, keepdims=True)
        shifted = out - m
        lse = jnp.log(jnp.sum(jnp.exp(shifted), axis=1, keepdims=True))
        o_ref[...] = (shifted - lse).astype(o_ref.dtype)


def _unit_schedule(edge_index, n_pad):
    e = edge_index.shape[1]
    n_t = n_pad // _T
    n_b = n_t * n_t
    g_units = e // _L + n_b + n_t + 1
    s_half = (e // _L + n_b + n_t * _U) // _U + 2
    q_half = s_half * _U

    src, dst = edge_index[0], edge_index[1]
    mask = jnp.int32(_T - 1)
    blk = (dst >> _TSHIFT) * n_t + (src >> _TSHIFT)
    key = (blk << (2 * _TSHIFT)) | ((dst & mask) << _TSHIFT) | (src & mask)
    ks = jnp.sort(key)

    bounds = (jnp.arange(n_b + 1, dtype=jnp.int32) << (2 * _TSHIFT))
    bnd = jnp.searchsorted(ks, bounds, side="left").astype(jnp.int32)
    blk_start = bnd[:-1]
    cnt = bnd[1:] - bnd[:-1]

    u_b = jnp.where(
        cnt > 0,
        ((blk_start + cnt - 1) >> 7) - (blk_start >> 7) + 1,
        0).astype(jnp.int32)
    per_tile = u_b.reshape(n_t, n_t)
    fix = (per_tile.sum(axis=1) == 0).astype(jnp.int32)
    col0 = (jnp.arange(n_t, dtype=jnp.int32)[None, :] == 0).astype(jnp.int32)
    u_b = (per_tile + fix[:, None] * col0).reshape(-1)

    u_excl = jnp.concatenate(
        [jnp.zeros((1,), jnp.int32), jnp.cumsum(u_b).astype(jnp.int32)])
    uidx = jnp.arange(g_units, dtype=jnp.int32)
    blk_of = jnp.minimum(jnp.searchsorted(u_excl[1:], uidx, side="right"
                                          ).astype(jnp.int32), n_b - 1)
    rank = uidx - u_excl[blk_of]
    row_u = (blk_start[blk_of] >> 7) + rank
    lo_u = jnp.clip(blk_start[blk_of] - (row_u << 7), 0, _L)
    hi_u = jnp.clip(blk_start[blk_of] + cnt[blk_of] - (row_u << 7), 0, _L)
    tile_u = blk_of // n_t
    kblk_u = blk_of % n_t
    m1_u = (row_u << 10) | (kblk_u << 5) | tile_u
    m2_u = (lo_u << 2) | (hi_u << 9)

    m_i = u_b.reshape(n_t, n_t).sum(axis=1)
    pm_i = ((m_i + _U - 1) // _U) * _U
    p_cum = jnp.cumsum(pm_i).astype(jnp.int32)
    p_excl = jnp.concatenate([jnp.zeros((1,), jnp.int32), p_cum])
    r_excl = jnp.concatenate(
        [jnp.zeros((1,), jnp.int32), jnp.cumsum(m_i).astype(jnp.int32)])
    qp_total = p_cum[-1]

    s = jnp.clip(jnp.searchsorted(p_cum, qp_total // 2, side="left"),
                 0, n_t - 2).astype(jnp.int32)
    cs = p_cum[s]

    gq = jnp.arange(q_half, dtype=jnp.int32)
    idx0 = jnp.clip(gq, 0, cs - 1)
    idx1 = jnp.clip(cs + gq, 0, qp_total - 1)
    real = jnp.stack([gq < cs, (cs + gq) < qp_total])
    q = jnp.stack([idx0, idx1])

    tile_q = jnp.minimum(
        jnp.searchsorted(p_cum, q.reshape(-1), side="right"
                         ).astype(jnp.int32), n_t - 1).reshape(2, q_half)
    rank_q = q - p_excl[tile_q]
    is_real = real & (rank_q < m_i[tile_q])
    src_idx = jnp.clip(r_excl[tile_q] + rank_q, 0, g_units - 1)
    m1 = jnp.where(is_real, m1_u[src_idx], tile_q)
    m2 = jnp.where(is_real, m2_u[src_idx], 0)

    valid = ((m2 >> 9) > ((m2 >> 2) & 0x7F)).astype(jnp.int32)
    nval = valid.reshape(2, s_half, _U).sum(axis=2)
    tile_s = tile_q.reshape(2, s_half, _U)[:, :, 0]
    first_s = jnp.concatenate(
        [jnp.ones((2, 1), jnp.int32),
         (tile_s[:, 1:] != tile_s[:, :-1]).astype(jnp.int32)], axis=1)
    last_s = jnp.concatenate(
        [(tile_s[:, 1:] != tile_s[:, :-1]).astype(jnp.int32),
         jnp.ones((2, 1), jnp.int32)], axis=1)
    nv = nval | (first_s << 4) | (last_s << 5)

    r_real = max((e + _L - 1) // _L, 1)
    r_pad = _round_up(r_real, 8) + 8
    ks2 = jnp.pad(ks, (0, r_pad * _L - e)).reshape(r_pad, _L)

    return (m1.astype(jnp.int32), m2.astype(jnp.int32), nv.astype(jnp.int32),
            ks2, s_half)


def kernel(x, edge_index, conv0_w_l, conv0_w_r, conv0_b_l,
           out_w_l, out_w_r, out_b_l):
    n, f_in = x.shape
    f_hid = conv0_w_l.shape[1]
    n_classes = out_w_l.shape[1]

    n_pad = _round_up(n, _T)
    f_in_p = _round_up(f_in, 128)
    f_hid_p = _round_up(f_hid, 128)
    f_out_p = _round_up(n_classes, 128)
    n_rows = n_pad // _T

    m1, m2, nv, ks2, s_half = _unit_schedule(edge_index, n_pad)
    r_pad = ks2.shape[0]

    xb = _pad2d(x, n_pad, f_in_p).astype(jnp.bfloat16)
    w1l = _pad2d(conv0_w_l, f_in_p, f_hid_p).astype(jnp.bfloat16)
    w1r = _pad2d(conv0_w_r, f_in_p, f_hid_p).astype(jnp.bfloat16)
    b1 = _pad2d(conv0_b_l, 1, f_hid_p)
    w2l = _pad2d(out_w_l, f_hid_p, f_out_p).astype(jnp.bfloat16)
    w2r = _pad2d(out_w_r, f_hid_p, f_out_p).astype(jnp.bfloat16)
    b2 = _pad2d(out_b_l, 1, f_out_p)

    h1p = pl.pallas_call(
        _proj_kernel,
        out_shape=jax.ShapeDtypeStruct((n_pad, f_hid_p), jnp.bfloat16),
        grid=(n_rows,),
        in_specs=[
            pl.BlockSpec((_T, f_in_p), lambda i: (i, 0)),
            pl.BlockSpec((f_in_p, f_hid_p), lambda i: (0, 0)),
        ],
        out_specs=pl.BlockSpec((_T, f_hid_p), lambda i: (i, 0)),
        compiler_params=pltpu.CompilerParams(
            dimension_semantics=("parallel",)),
    )(xb, w1l)

    cparams = pltpu.CompilerParams(
        dimension_semantics=("parallel", "arbitrary"),
        vmem_limit_bytes=48 * 1024 * 1024,
    )

    h1, h2p, inv_deg = pl.pallas_call(
        _agg1_kernel,
        out_shape=(
            jax.ShapeDtypeStruct((n_pad, f_hid_p), jnp.bfloat16),
            jax.ShapeDtypeStruct((n_pad, f_out_p), jnp.bfloat16),
            jax.ShapeDtypeStruct((n_pad, 1), jnp.float32),
        ),
        grid_spec=pltpu.PrefetchScalarGridSpec(
            num_scalar_prefetch=3,
            grid=(2, s_half),
            in_specs=[
                pl.BlockSpec((r_pad, _L), lambda c, g, m1r, m2r, nvr: (0, 0)),
                pl.BlockSpec((n_pad, f_hid_p), lambda c, g, m1r, m2r, nvr: (0, 0)),
                pl.BlockSpec((_T, f_in_p),
                             lambda c, g, m1r, m2r, nvr: (m1r[c, g * _U] & 31, 0)),
                pl.BlockSpec((f_in_p, f_hid_p), lambda c, g, m1r, m2r, nvr: (0, 0)),
                pl.BlockSpec((1, f_hid_p), lambda c, g, m1r, m2r, nvr: (0, 0)),
                pl.BlockSpec((f_hid_p, f_out_p), lambda c, g, m1r, m2r, nvr: (0, 0)),
            ],
            out_specs=(
                pl.BlockSpec((_T, f_hid_p),
                             lambda c, g, m1r, m2r, nvr: (m1r[c, g * _U] & 31, 0)),
                pl.BlockSpec((_T, f_out_p),
                             lambda c, g, m1r, m2r, nvr: (m1r[c, g * _U] & 31, 0)),
                pl.BlockSpec((_T, 1),
                             lambda c, g, m1r, m2r, nvr: (m1r[c, g * _U] & 31, 0)),
            ),
            scratch_shapes=[pltpu.VMEM((_T, f_hid_p), jnp.float32),
                            pltpu.VMEM((_T, 1), jnp.float32)],
        ),
        compiler_params=cparams,
    )(m1, m2, nv, ks2, h1p, xb, w1r, b1, w2l)

    out = pl.pallas_call(
        functools.partial(_agg2_kernel, n_classes=n_classes),
        out_shape=jax.ShapeDtypeStruct((n_pad, f_out_p), jnp.float32),
        grid_spec=pltpu.PrefetchScalarGridSpec(
            num_scalar_prefetch=3,
            grid=(2, s_half),
            in_specs=[
                pl.BlockSpec((r_pad, _L), lambda c, g, m1r, m2r, nvr: (0, 0)),
                pl.BlockSpec((n_pad, f_out_p), lambda c, g, m1r, m2r, nvr: (0, 0)),
                pl.BlockSpec((_T, f_hid_p),
                             lambda c, g, m1r, m2r, nvr: (m1r[c, g * _U] & 31, 0)),
                pl.BlockSpec((f_hid_p, f_out_p), lambda c, g, m1r, m2r, nvr: (0, 0)),
                pl.BlockSpec((1, f_out_p), lambda c, g, m1r, m2r, nvr: (0, 0)),
                pl.BlockSpec((_T, 1),
                             lambda c, g, m1r, m2r, nvr: (m1r[c, g * _U] & 31, 0)),
            ],
            out_specs=pl.BlockSpec((_T, f_out_p),
                                   lambda c, g, m1r, m2r, nvr: (m1r[c, g * _U] & 31, 0)),
            scratch_shapes=[pltpu.VMEM((_T, f_out_p), jnp.float32)],
        ),
        compiler_params=cparams,
    )(m1, m2, nv, ks2, h2p, h1, w2r, b2, inv_deg)

    return out[:n, :n_classes]


# speedup vs baseline: 12.6984x; 1.6834x over previous
import functools

import jax
import jax.numpy as jnp
from jax.experimental import pallas as pl
from jax.experimental.pallas import tpu as pltpu

_T = 512
_TSHIFT = 9
_L = 128
_U = 16


def _round_up(x, m):
    return ((x + m - 1) // m) * m


def _pad2d(a, rows, cols):
    if a.shape == (rows, cols):
        return a
    return jnp.pad(a, ((0, rows - a.shape[0]), (0, cols - a.shape[1])))


def _proj_kernel(x_ref, w_ref, h_ref):
    h_ref[...] = jnp.dot(x_ref[...], w_ref[...],
                         preferred_element_type=jnp.float32).astype(h_ref.dtype)


def _unit_onehots(ks_ref, m1, m2):
    row = m1 >> 10
    lo = (m2 >> 2) & 0x7F
    hi = (m2 >> 9) & 0xFF
    base = pl.multiple_of((row >> 3) << 3, 8)
    slab = ks_ref[pl.ds(base, 8), :]
    kv = pltpu.roll(slab, -(row & 7), axis=0)[0:1, :]
    lane = jax.lax.broadcasted_iota(jnp.int32, (1, _L), 1)
    msk = (lane >= lo) & (lane < hi)
    mask_i = jnp.int32(_T - 1)
    srcv = jnp.where(msk, kv & mask_i, -1)
    dstv = jnp.where(msk, (kv >> _TSHIFT) & mask_i, -1)
    rowio = jax.lax.broadcasted_iota(jnp.int32, (_T, _L), 0)
    oh_s = (rowio == srcv).astype(jnp.bfloat16)
    oh_d = (rowio == dstv).astype(jnp.bfloat16)
    return oh_s, oh_d


def _gather_scatter(ks_ref, hp_ref, m1, m2):
    oh_s, oh_d = _unit_onehots(ks_ref, m1, m2)
    koff = pl.multiple_of(((m1 >> 5) & 31) * _T, _T)
    grows = jax.lax.dot_general(
        oh_s, hp_ref[pl.ds(koff, _T), :],
        dimension_numbers=(((0,), (0,)), ((), ())),
        preferred_element_type=jnp.float32)
    contrib = jnp.dot(oh_d, grows.astype(jnp.bfloat16),
                      preferred_element_type=jnp.float32)
    return contrib, oh_d


def _agg1_kernel(m1_r, m2_r, nv_r, ks_ref, hp_ref, x_ref, wr_ref, b_ref,
                 w2_ref, h1_ref, h2p_ref, invd_ref, acc_ref, dacc_ref):
    c = pl.program_id(0)
    g = pl.program_id(1)
    nv = nv_r[c, g]

    @pl.when(nv & 16 == 16)
    def _():
        acc_ref[...] = jnp.zeros_like(acc_ref)
        dacc_ref[...] = jnp.zeros_like(dacc_ref)

    @pl.when(nv & 15 > 0)
    def _():
        total = None
        drow = None
        for u in range(_U):
            m1 = m1_r[c, g * _U + u]
            m2 = m2_r[c, g * _U + u]
            contrib, oh_d = _gather_scatter(ks_ref, hp_ref, m1, m2)
            d = jnp.sum(oh_d, axis=1, keepdims=True).astype(jnp.float32)
            total = contrib if total is None else total + contrib
            drow = d if drow is None else drow + d
        acc_ref[...] += total
        dacc_ref[...] += drow

    @pl.when(nv & 32 == 32)
    def _():
        deg = dacc_ref[...]
        inv = jnp.where(deg > 0, 1.0 / deg, 0.0)
        invd_ref[...] = inv
        self_term = jnp.dot(x_ref[...], wr_ref[...],
                            preferred_element_type=jnp.float32) + b_ref[...]
        h1 = jnp.maximum(acc_ref[...] * inv + self_term, 0.0)
        h1_bf = h1.astype(jnp.bfloat16)
        h1_ref[...] = h1_bf
        h2p_ref[...] = jnp.dot(h1_bf, w2_ref[...],
                               preferred_element_type=jnp.float32
                               ).astype(h2p_ref.dtype)


def _agg2_kernel(m1_r, m2_r, nv_r, ks_ref, hp_ref, h1_ref, wr_ref, b_ref,
                 inv_ref, o_ref, acc_ref, *, n_classes):
    c = pl.program_id(0)
    g = pl.program_id(1)
    nv = nv_r[c, g]

    @pl.when(nv & 16 == 16)
    def _():
        acc_ref[...] = jnp.zeros_like(acc_ref)

    @pl.when(nv & 15 > 0)
    def _():
        total = None
        for u in range(_U):
            m1 = m1_r[c, g * _U + u]
            m2 = m2_r[c, g * _U + u]
            contrib, _ = _gather_scatter(ks_ref, hp_ref, m1, m2)
            total = contrib if total is None else total + contrib
        acc_ref[...] += total

    @pl.when(nv & 32 == 32)
    def _():
        self_term = jnp.dot(h1_ref[...], wr_ref[...],
                            preferred_element_type=jnp.float32) + b_ref[...]
        out = acc_ref[...] * inv_ref[...] + self_term
        col = jax.lax.broadcasted_iota(jnp.int32, out.shape, 1)
        out = jnp.where(col < n_classes, out, -jnp.inf)
        m = jnp.max(out, axis=1, keepdims=True)
        shifted = out - m
        lse = jnp.log(jnp.sum(jnp.exp(shifted), axis=1, keepdims=True))
        o_ref[...] = (shifted - lse).astype(o_ref.dtype)


def _unit_schedule(edge_index, n_pad):
    e = edge_index.shape[1]
    n_t = n_pad // _T
    n_b = n_t * n_t
    g_units = e // _L + n_b + n_t + 1
    s_half = (e // _L + n_b + n_t * _U) // _U + 2
    q_half = s_half * _U

    src, dst = edge_index[0], edge_index[1]
    mask = jnp.int32(_T - 1)
    blk = (dst >> _TSHIFT) * n_t + (src >> _TSHIFT)
    key = (blk << (2 * _TSHIFT)) | ((dst & mask) << _TSHIFT) | (src & mask)
    ks = jnp.sort(key)

    bounds = (jnp.arange(n_b + 1, dtype=jnp.int32) << (2 * _TSHIFT))
    bnd = jnp.searchsorted(ks, bounds, side="left").astype(jnp.int32)
    blk_start = bnd[:-1]
    cnt = bnd[1:] - bnd[:-1]

    u_b = jnp.where(
        cnt > 0,
        ((blk_start + cnt - 1) >> 7) - (blk_start >> 7) + 1,
        0).astype(jnp.int32)
    per_tile = u_b.reshape(n_t, n_t)
    fix = (per_tile.sum(axis=1) == 0).astype(jnp.int32)
    col0 = (jnp.arange(n_t, dtype=jnp.int32)[None, :] == 0).astype(jnp.int32)
    u_b = (per_tile + fix[:, None] * col0).reshape(-1)

    u_excl = jnp.concatenate(
        [jnp.zeros((1,), jnp.int32), jnp.cumsum(u_b).astype(jnp.int32)])
    uidx = jnp.arange(g_units, dtype=jnp.int32)
    blk_of = jnp.minimum(jnp.searchsorted(u_excl[1:], uidx, side="right"
                                          ).astype(jnp.int32), n_b - 1)
    rank = uidx - u_excl[blk_of]
    row_u = (blk_start[blk_of] >> 7) + rank
    lo_u = jnp.clip(blk_start[blk_of] - (row_u << 7), 0, _L)
    hi_u = jnp.clip(blk_start[blk_of] + cnt[blk_of] - (row_u << 7), 0, _L)
    tile_u = blk_of // n_t
    kblk_u = blk_of % n_t
    m1_u = (row_u << 10) | (kblk_u << 5) | tile_u
    m2_u = (lo_u << 2) | (hi_u << 9)

    m_i = u_b.reshape(n_t, n_t).sum(axis=1)
    pm_i = ((m_i + _U - 1) // _U) * _U
    p_cum = jnp.cumsum(pm_i).astype(jnp.int32)
    p_excl = jnp.concatenate([jnp.zeros((1,), jnp.int32), p_cum])
    r_excl = jnp.concatenate(
        [jnp.zeros((1,), jnp.int32), jnp.cumsum(m_i).astype(jnp.int32)])
    qp_total = p_cum[-1]

    s = jnp.clip(jnp.searchsorted(p_cum, qp_total // 2, side="left"),
                 0, n_t - 2).astype(jnp.int32)
    cs = p_cum[s]

    gq = jnp.arange(q_half, dtype=jnp.int32)
    idx0 = jnp.clip(gq, 0, cs - 1)
    idx1 = jnp.clip(cs + gq, 0, qp_total - 1)
    real = jnp.stack([gq < cs, (cs + gq) < qp_total])
    q = jnp.stack([idx0, idx1])

    tile_q = jnp.minimum(
        jnp.searchsorted(p_cum, q.reshape(-1), side="right"
                         ).astype(jnp.int32), n_t - 1).reshape(2, q_half)
    rank_q = q - p_excl[tile_q]
    is_real = real & (rank_q < m_i[tile_q])
    src_idx = jnp.clip(r_excl[tile_q] + rank_q, 0, g_units - 1)
    m1 = jnp.where(is_real, m1_u[src_idx], tile_q)
    m2 = jnp.where(is_real, m2_u[src_idx], 0)

    valid = ((m2 >> 9) > ((m2 >> 2) & 0x7F)).astype(jnp.int32)
    nval = valid.reshape(2, s_half, _U).sum(axis=2)
    tile_s = tile_q.reshape(2, s_half, _U)[:, :, 0]
    first_s = jnp.concatenate(
        [jnp.ones((2, 1), jnp.int32),
         (tile_s[:, 1:] != tile_s[:, :-1]).astype(jnp.int32)], axis=1)
    last_s = jnp.concatenate(
        [(tile_s[:, 1:] != tile_s[:, :-1]).astype(jnp.int32),
         jnp.ones((2, 1), jnp.int32)], axis=1)
    nv = nval | (first_s << 4) | (last_s << 5)

    r_real = max((e + _L - 1) // _L, 1)
    r_pad = _round_up(r_real, 8) + 8
    ks2 = jnp.pad(ks, (0, r_pad * _L - e)).reshape(r_pad, _L)

    return (m1.astype(jnp.int32), m2.astype(jnp.int32), nv.astype(jnp.int32),
            ks2, s_half)


def kernel(x, edge_index, conv0_w_l, conv0_w_r, conv0_b_l,
           out_w_l, out_w_r, out_b_l):
    n, f_in = x.shape
    f_hid = conv0_w_l.shape[1]
    n_classes = out_w_l.shape[1]

    n_pad = _round_up(n, _T)
    f_in_p = _round_up(f_in, 128)
    f_hid_p = _round_up(f_hid, 128)
    f_out_p = _round_up(n_classes, 128)
    n_rows = n_pad // _T

    m1, m2, nv, ks2, s_half = _unit_schedule(edge_index, n_pad)
    r_pad = ks2.shape[0]

    xb = _pad2d(x, n_pad, f_in_p).astype(jnp.bfloat16)
    w1l = _pad2d(conv0_w_l, f_in_p, f_hid_p).astype(jnp.bfloat16)
    w1r = _pad2d(conv0_w_r, f_in_p, f_hid_p).astype(jnp.bfloat16)
    b1 = _pad2d(conv0_b_l, 1, f_hid_p)
    w2l = _pad2d(out_w_l, f_hid_p, f_out_p).astype(jnp.bfloat16)
    w2r = _pad2d(out_w_r, f_hid_p, f_out_p).astype(jnp.bfloat16)
    b2 = _pad2d(out_b_l, 1, f_out_p)

    h1p = pl.pallas_call(
        _proj_kernel,
        out_shape=jax.ShapeDtypeStruct((n_pad, f_hid_p), jnp.bfloat16),
        grid=(n_rows,),
        in_specs=[
            pl.BlockSpec((_T, f_in_p), lambda i: (i, 0)),
            pl.BlockSpec((f_in_p, f_hid_p), lambda i: (0, 0)),
        ],
        out_specs=pl.BlockSpec((_T, f_hid_p), lambda i: (i, 0)),
        compiler_params=pltpu.CompilerParams(
            dimension_semantics=("parallel",)),
    )(xb, w1l)

    cparams = pltpu.CompilerParams(
        dimension_semantics=("parallel", "arbitrary"),
        vmem_limit_bytes=48 * 1024 * 1024,
    )

    h1, h2p, inv_deg = pl.pallas_call(
        _agg1_kernel,
        out_shape=(
            jax.ShapeDtypeStruct((n_pad, f_hid_p), jnp.bfloat16),
            jax.ShapeDtypeStruct((n_pad, f_out_p), jnp.bfloat16),
            jax.ShapeDtypeStruct((n_pad, 1), jnp.float32),
        ),
        grid_spec=pltpu.PrefetchScalarGridSpec(
            num_scalar_prefetch=3,
            grid=(2, s_half),
            in_specs=[
                pl.BlockSpec((r_pad, _L), lambda c, g, m1r, m2r, nvr: (0, 0)),
                pl.BlockSpec((n_pad, f_hid_p), lambda c, g, m1r, m2r, nvr: (0, 0)),
                pl.BlockSpec((_T, f_in_p),
                             lambda c, g, m1r, m2r, nvr: (m1r[c, g * _U] & 31, 0)),
                pl.BlockSpec((f_in_p, f_hid_p), lambda c, g, m1r, m2r, nvr: (0, 0)),
                pl.BlockSpec((1, f_hid_p), lambda c, g, m1r, m2r, nvr: (0, 0)),
                pl.BlockSpec((f_hid_p, f_out_p), lambda c, g, m1r, m2r, nvr: (0, 0)),
            ],
            out_specs=(
                pl.BlockSpec((_T, f_hid_p),
                             lambda c, g, m1r, m2r, nvr: (m1r[c, g * _U] & 31, 0)),
                pl.BlockSpec((_T, f_out_p),
                             lambda c, g, m1r, m2r, nvr: (m1r[c, g * _U] & 31, 0)),
                pl.BlockSpec((_T, 1),
                             lambda c, g, m1r, m2r, nvr: (m1r[c, g * _U] & 31, 0)),
            ),
            scratch_shapes=[pltpu.VMEM((_T, f_hid_p), jnp.float32),
                            pltpu.VMEM((_T, 1), jnp.float32)],
        ),
        compiler_params=cparams,
    )(m1, m2, nv, ks2, h1p, xb, w1r, b1, w2l)

    out = pl.pallas_call(
        functools.partial(_agg2_kernel, n_classes=n_classes),
        out_shape=jax.ShapeDtypeStruct((n_pad, f_out_p), jnp.float32),
        grid_spec=pltpu.PrefetchScalarGridSpec(
            num_scalar_prefetch=3,
            grid=(2, s_half),
            in_specs=[
                pl.BlockSpec((r_pad, _L), lambda c, g, m1r, m2r, nvr: (0, 0)),
                pl.BlockSpec((n_pad, f_out_p), lambda c, g, m1r, m2r, nvr: (0, 0)),
                pl.BlockSpec((_T, f_hid_p),
                             lambda c, g, m1r, m2r, nvr: (m1r[c, g * _U] & 31, 0)),
                pl.BlockSpec((f_hid_p, f_out_p), lambda c, g, m1r, m2r, nvr: (0, 0)),
                pl.BlockSpec((1, f_out_p), lambda c, g, m1r, m2r, nvr: (0, 0)),
                pl.BlockSpec((_T, 1),
                             lambda c, g, m1r, m2r, nvr: (m1r[c, g * _U] & 31, 0)),
            ],
            out_specs=pl.BlockSpec((_T, f_out_p),
                                   lambda c, g, m1r, m2r, nvr: (m1r[c, g * _U] & 31, 0)),
            scratch_shapes=[pltpu.VMEM((_T, f_out_p), jnp.float32)],
        ),
        compiler_params=cparams,
    )(m1, m2, nv, ks2, h2p, h1, w2r, b2, inv_deg)

    return out[:n, :n_classes]
